# baseline (speedup 1.0000x reference)
import functools

import jax
import jax.numpy as jnp
import numpy as np
from jax import lax
from jax.experimental import pallas as pl
from jax.experimental.pallas import tpu as pltpu
from jax.experimental.pallas import tpu_sc as plsc


_BIG_I32 = np.int32(2**30)


_SL = 16


def _argmin_body(k_chunk, n_chunks, x_ref, l2xh_ref, e_ref, l2eh_ref, codes_ref):
    m_t = x_ref.shape[0]
    sl = _SL
    n_sl = k_chunk // sl
    x = x_ref[...]
    l2xh = l2xh_ref[0]
    iota_sub = lax.broadcasted_iota(jnp.int32, (sl, m_t), 0).astype(jnp.float32)

    rmin = jnp.full((sl, m_t), jnp.inf, jnp.float32)
    rix = jnp.zeros((sl, m_t), jnp.float32)
    for k in range(n_chunks):
        e = e_ref[pl.ds(k * k_chunk, k_chunk), :]
        l2eh = l2eh_ref[pl.ds(k * k_chunk, k_chunk), :]
        dot = lax.dot_general(e, x, (((1,), (1,)), ((), ())),
                              preferred_element_type=jnp.float32)
        for s in range(n_sl):
            d = lax.slice(dot, (s * sl, 0), ((s + 1) * sl, m_t))
            l2eh_s = lax.slice(l2eh, (s * sl, 0), ((s + 1) * sl, 1))
            dist = (l2xh + l2eh_s) - d
            mask = dist < rmin
            rmin = jnp.minimum(rmin, dist)
            rix = jnp.where(mask, np.float32(k * n_sl + s), rix)
    gmin = jnp.min(rmin, axis=0, keepdims=True)
    gidx = rix * np.float32(sl) + iota_sub
    cand = jnp.where(rmin == gmin, gidx, np.float32(65536.0))
    best = jnp.min(cand, axis=0, keepdims=True)
    codes_ref[...] = best.astype(jnp.int32)[None]


def _codes_tc(x, l2xh3, emb, l2eh2, m_t=2048, k_chunk=4096):
    m, d = x.shape
    k, _ = emb.shape
    n_tiles = m // m_t
    n_chunks = k // k_chunk
    body = functools.partial(_argmin_body, k_chunk, n_chunks)
    return pl.pallas_call(
        body,
        grid=(n_tiles,),
        in_specs=[
            pl.BlockSpec((m_t, d), lambda i: (i, 0)),
            pl.BlockSpec((1, 1, m_t), lambda i: (i, 0, 0)),
            pl.BlockSpec((k, d), lambda i: (0, 0)),
            pl.BlockSpec((k, 1), lambda i: (0, 0)),
        ],
        out_specs=pl.BlockSpec((1, 1, m_t), lambda i: (i, 0, 0)),
        out_shape=jax.ShapeDtypeStruct((n_tiles, 1, m_t), jnp.int32),
    )(x, l2xh3, emb, l2eh2)


def _gather_sc(emb, codes_flat):
    k, d = emb.shape
    b = codes_flat.shape[0]
    info = plsc.get_sparse_core_info()
    nw = info.num_cores * info.num_subcores
    bpw = b // nw
    chunk = 128
    n_chunks = bpw // chunk
    mesh = plsc.VectorSubcoreMesh(core_axis_name="c", subcore_axis_name="s")

    @functools.partial(
        pl.kernel, mesh=mesh,
        out_type=jax.ShapeDtypeStruct((b, d), jnp.float32),
        scratch_types=[
            pltpu.VMEM((chunk,), jnp.int32),
            pltpu.VMEM((chunk, d), jnp.float32),
            pltpu.SemaphoreType.DMA,
        ],
    )
    def gather(emb_hbm, codes_hbm, out_hbm, idx_v, rows_v, sem):
        wid = lax.axis_index("s") * info.num_cores + lax.axis_index("c")
        base = wid * bpw
        for c in range(n_chunks):
            off = base + c * chunk
            pltpu.sync_copy(codes_hbm.at[pl.ds(off, chunk)], idx_v)
            pltpu.async_copy(emb_hbm.at[idx_v], rows_v, sem).wait()
            pltpu.sync_copy(rows_v, out_hbm.at[pl.ds(off, chunk)])

    return gather(emb, codes_flat)


def kernel(inputs, embeddings):
    b, h, w, d = inputs.shape
    m = b * h * w
    x = inputs.reshape(m, d)
    l2xh = 0.5 * jnp.sum(inputs ** 2, axis=-1, keepdims=True)
    l2eh = (0.5 * jnp.sum(embeddings ** 2, axis=-1)).reshape(-1, 1)

    m_t = 2048
    half = m // 2
    codes, vecs = [], []
    for lo in (0, half):
        xh = lax.slice(x, (lo, 0), (lo + half, d))
        l2xh3 = lax.slice(l2xh.reshape(1, m), (0, lo), (1, lo + half))
        c = _codes_tc(xh, l2xh3.reshape(half // m_t, 1, m_t), embeddings,
                      l2eh, m_t=m_t).reshape(half)
        codes.append(c)
        vecs.append(_gather_sc(embeddings, c))
    codes_flat = jnp.concatenate(codes)
    code_vecs = jnp.concatenate(vecs)
    return codes_flat.reshape(b, h, w), code_vecs.reshape(b, h, w, d)


# speedup vs baseline: 1.1924x; 1.1924x over previous
import functools

import jax
import jax.numpy as jnp
import numpy as np
from jax import lax
from jax.experimental import pallas as pl
from jax.experimental.pallas import tpu as pltpu
from jax.experimental.pallas import tpu_sc as plsc


_BIG_I32 = np.int32(2**30)


_SL = 16


def _argmin_body(k_chunk, n_chunks, x_ref, l2xh_ref, e_ref, l2eh_ref, codes_ref):
    m_t = x_ref.shape[0]
    sl = _SL
    n_sl = k_chunk // sl
    x = x_ref[...]
    l2xh = l2xh_ref[0]
    iota_sub = lax.broadcasted_iota(jnp.int32, (sl, m_t), 0).astype(jnp.float32)

    rmin = jnp.full((sl, m_t), jnp.inf, jnp.float32)
    rix = jnp.zeros((sl, m_t), jnp.float32)
    for k in range(n_chunks):
        e = e_ref[pl.ds(k * k_chunk, k_chunk), :]
        l2eh = l2eh_ref[pl.ds(k * k_chunk, k_chunk), :]
        dot = lax.dot_general(e, x, (((1,), (1,)), ((), ())),
                              preferred_element_type=jnp.float32)
        for s in range(n_sl):
            d = lax.slice(dot, (s * sl, 0), ((s + 1) * sl, m_t))
            l2eh_s = lax.slice(l2eh, (s * sl, 0), ((s + 1) * sl, 1))
            dist = (l2xh + l2eh_s) - d
            mask = dist < rmin
            rmin = jnp.minimum(rmin, dist)
            rix = jnp.where(mask, np.float32(k * n_sl + s), rix)
    gmin = jnp.min(rmin, axis=0, keepdims=True)
    gidx = rix * np.float32(sl) + iota_sub
    cand = jnp.where(rmin == gmin, gidx, np.float32(65536.0))
    best = jnp.min(cand, axis=0, keepdims=True)
    codes_ref[...] = best.astype(jnp.int32)[None]


def _codes_tc(x, l2xh3, emb, l2eh2, m_t=2048, k_chunk=4096):
    m, d = x.shape
    k, _ = emb.shape
    n_tiles = m // m_t
    n_chunks = k // k_chunk
    body = functools.partial(_argmin_body, k_chunk, n_chunks)
    return pl.pallas_call(
        body,
        grid=(n_tiles,),
        in_specs=[
            pl.BlockSpec((m_t, d), lambda i: (i, 0)),
            pl.BlockSpec((1, 1, m_t), lambda i: (i, 0, 0)),
            pl.BlockSpec((k, d), lambda i: (0, 0)),
            pl.BlockSpec((k, 1), lambda i: (0, 0)),
        ],
        out_specs=pl.BlockSpec((1, 1, m_t), lambda i: (i, 0, 0)),
        out_shape=jax.ShapeDtypeStruct((n_tiles, 1, m_t), jnp.int32),
    )(x, l2xh3, emb, l2eh2)


def _gather_sc(emb, codes_flat):
    k, d = emb.shape
    b = codes_flat.shape[0]
    info = plsc.get_sparse_core_info()
    nw = info.num_cores * info.num_subcores
    bpw = b // nw
    chunk = 128
    n_chunks = bpw // chunk
    mesh = plsc.VectorSubcoreMesh(core_axis_name="c", subcore_axis_name="s")

    @functools.partial(
        pl.kernel, mesh=mesh,
        out_type=jax.ShapeDtypeStruct((b, d), jnp.float32),
        scratch_types=[
            pltpu.VMEM((chunk,), jnp.int32),
            pltpu.VMEM((chunk, d), jnp.float32),
            pltpu.SemaphoreType.DMA,
        ],
    )
    def gather(emb_hbm, codes_hbm, out_hbm, idx_v, rows_v, sem):
        wid = lax.axis_index("s") * info.num_cores + lax.axis_index("c")
        base = wid * bpw
        for c in range(n_chunks):
            off = base + c * chunk
            pltpu.sync_copy(codes_hbm.at[pl.ds(off, chunk)], idx_v)
            pltpu.async_copy(emb_hbm.at[idx_v], rows_v, sem).wait()
            pltpu.sync_copy(rows_v, out_hbm.at[pl.ds(off, chunk)])

    return gather(emb, codes_flat)


def kernel(inputs, embeddings):
    b, h, w, d = inputs.shape
    m = b * h * w
    x = inputs.reshape(m, d)
    l2xh = 0.5 * jnp.sum(inputs ** 2, axis=-1, keepdims=True)
    l2eh = (0.5 * jnp.sum(embeddings ** 2, axis=-1)).reshape(-1, 1)

    m_t = 2048
    codes_flat = _codes_tc(x, l2xh.reshape(m // m_t, 1, m_t), embeddings,
                           l2eh, m_t=m_t).reshape(m)
    code_vecs = _gather_sc(embeddings, codes_flat)
    return codes_flat.reshape(b, h, w), code_vecs.reshape(b, h, w, d)


# speedup vs baseline: 1.2122x; 1.0166x over previous
import functools

import jax
import jax.numpy as jnp
import numpy as np
from jax import lax
from jax.experimental import pallas as pl
from jax.experimental.pallas import tpu as pltpu
from jax.experimental.pallas import tpu_sc as plsc


_BIG_I32 = np.int32(2**30)


_SL = 16


def _argmin_body(k_chunk, n_chunks, x_ref, l2xh_ref, e_ref, l2eh_ref, codes_ref):
    m_t = x_ref.shape[0]
    sl = _SL
    n_sl = k_chunk // sl
    x = x_ref[...]
    l2xh = l2xh_ref[0]
    iota_sub = lax.broadcasted_iota(jnp.int32, (sl, m_t), 0).astype(jnp.float32)

    rmin = jnp.full((sl, m_t), jnp.inf, jnp.float32)
    rix = jnp.zeros((sl, m_t), jnp.float32)
    for k in range(n_chunks):
        e = e_ref[pl.ds(k * k_chunk, k_chunk), :]
        l2eh = l2eh_ref[pl.ds(k * k_chunk, k_chunk), :]
        dot = lax.dot_general(e, x, (((1,), (1,)), ((), ())),
                              preferred_element_type=jnp.float32)
        for s in range(n_sl):
            d = lax.slice(dot, (s * sl, 0), ((s + 1) * sl, m_t))
            l2eh_s = lax.slice(l2eh, (s * sl, 0), ((s + 1) * sl, 1))
            dist = (l2xh + l2eh_s) - d
            mask = dist < rmin
            rmin = jnp.minimum(rmin, dist)
            rix = jnp.where(mask, np.float32(k * n_sl + s), rix)
    gmin = jnp.min(rmin, axis=0, keepdims=True)
    gidx = rix * np.float32(sl) + iota_sub
    cand = jnp.where(rmin == gmin, gidx, np.float32(65536.0))
    best = jnp.min(cand, axis=0, keepdims=True)
    codes_ref[...] = best.astype(jnp.int32)[None]


def _codes_tc(x, l2xh3, emb, l2eh2, m_t=2048, k_chunk=4096):
    m, d = x.shape
    k, _ = emb.shape
    n_tiles = m // m_t
    n_chunks = k // k_chunk
    body = functools.partial(_argmin_body, k_chunk, n_chunks)
    return pl.pallas_call(
        body,
        grid=(n_tiles,),
        in_specs=[
            pl.BlockSpec((m_t, d), lambda i: (i, 0)),
            pl.BlockSpec((1, 1, m_t), lambda i: (i, 0, 0)),
            pl.BlockSpec((k, d), lambda i: (0, 0)),
            pl.BlockSpec((k, 1), lambda i: (0, 0)),
        ],
        out_specs=pl.BlockSpec((1, 1, m_t), lambda i: (i, 0, 0)),
        out_shape=jax.ShapeDtypeStruct((n_tiles, 1, m_t), jnp.int32),
    )(x, l2xh3, emb, l2eh2)


def _gather_sc(emb, codes_flat):
    k, d = emb.shape
    b = codes_flat.shape[0]
    info = plsc.get_sparse_core_info()
    nw = info.num_cores * info.num_subcores
    bpw = b // nw
    chunk = 128
    n_chunks = bpw // chunk
    codes2 = codes_flat.reshape(b // chunk, chunk)
    mesh = plsc.VectorSubcoreMesh(core_axis_name="c", subcore_axis_name="s")

    @functools.partial(
        pl.kernel, mesh=mesh,
        out_type=jax.ShapeDtypeStruct((b, d), jnp.float32),
        scratch_types=[
            pltpu.VMEM((n_chunks, chunk), jnp.int32),
            pltpu.VMEM((chunk, d), jnp.float32),
            pltpu.VMEM((chunk, d), jnp.float32),
            pltpu.SemaphoreType.DMA,
            pltpu.SemaphoreType.DMA,
            pltpu.SemaphoreType.DMA,
            pltpu.SemaphoreType.DMA,
        ],
    )
    def gather(emb_hbm, codes_hbm, out_hbm, idx_v, rows0, rows1,
               sg0, sg1, sw0, sw1):
        wid = lax.axis_index("s") * info.num_cores + lax.axis_index("c")
        rows, sg, sw = (rows0, rows1), (sg0, sg1), (sw0, sw1)
        pltpu.sync_copy(codes_hbm.at[pl.ds(wid * n_chunks, n_chunks)], idx_v)

        def g(c):
            return pltpu.async_copy(emb_hbm.at[idx_v.at[c]], rows[c % 2],
                                    sg[c % 2])

        def wb(c):
            off = wid * bpw + c * chunk
            return pltpu.async_copy(rows[c % 2], out_hbm.at[pl.ds(off, chunk)],
                                    sw[c % 2])

        gops = {c: g(c) for c in range(min(2, n_chunks))}
        wops = {}
        for c in range(n_chunks):
            gops[c].wait()
            wops[c] = wb(c)
            if c + 2 < n_chunks:
                wops[c].wait()
                gops[c + 2] = g(c + 2)
        for c in range(max(0, n_chunks - 2), n_chunks):
            wops[c].wait()

    return gather(emb, codes2)


def kernel(inputs, embeddings):
    b, h, w, d = inputs.shape
    m = b * h * w
    x = inputs.reshape(m, d)
    l2xh = 0.5 * jnp.sum(inputs ** 2, axis=-1, keepdims=True)
    l2eh = (0.5 * jnp.sum(embeddings ** 2, axis=-1)).reshape(-1, 1)

    m_t = 2048
    codes_flat = _codes_tc(x, l2xh.reshape(m // m_t, 1, m_t), embeddings,
                           l2eh, m_t=m_t).reshape(m)
    code_vecs = _gather_sc(embeddings, codes_flat)
    return codes_flat.reshape(b, h, w), code_vecs.reshape(b, h, w, d)
